# baseline (speedup 1.0000x reference)
import functools

import jax
import jax.numpy as jnp
from jax import lax
from jax.experimental import pallas as pl
from jax.experimental.pallas import tpu as pltpu
from jax.experimental.pallas import tpu_sc as plsc

NUM_EMB = 100000
D = 128
HALF = D // 2
B = 16384
NSPLIT = 2
BS = B // NSPLIT
NC = 2
NS = 16
NW = NC * NS
BPW = BS // NW
CHUNK = 128
NCHUNK = BPW // CHUNK
RB = 2
OB = 2

_mesh = plsc.VectorSubcoreMesh(core_axis_name="c", subcore_axis_name="s")


@functools.partial(
    pl.kernel,
    mesh=_mesh,
    out_type=(jax.ShapeDtypeStruct((BS, HALF), jnp.float32),
              jax.ShapeDtypeStruct((BS, HALF), jnp.float32)),
    scratch_types=(
        [pltpu.VMEM((BPW,), jnp.int32)]
        + [pltpu.VMEM((CHUNK, D), jnp.float32) for _ in range(RB)]
        + [pltpu.VMEM((CHUNK, HALF), jnp.float32) for _ in range(2 * OB)]
        + [pltpu.SemaphoreType.DMA((NCHUNK,)), pltpu.SemaphoreType.DMA]
    ),
    compiler_params=pltpu.CompilerParams(needs_layout_passes=False),
)
def _gather_split(ids_hbm, table_hbm, re_hbm, im_hbm, idx_v,
                  rows0, rows1, re0, re1, im0, im1,
                  gsem, osem):
    rows = [rows0, rows1]
    res = [re0, re1]
    ims = [im0, im1]
    wid = lax.axis_index("s") * NC + lax.axis_index("c")
    base = wid * BPW
    pltpu.sync_copy(ids_hbm.at[pl.ds(base, BPW)], idx_v)

    def fire_gather(c):
        return pltpu.async_copy(
            table_hbm.at[idx_v.at[pl.ds(c * CHUNK, CHUNK)]],
            rows[c % RB], gsem.at[c])

    gathers = [None] * NCHUNK
    for c in range(min(RB, NCHUNK)):
        gathers[c] = fire_gather(c)

    evens = lax.iota(jnp.int32, 16) * 2
    cols = [evens + 32 * j for j in range(HALF // 16)]
    cols1 = [col + 1 for col in cols]

    writes = [None] * (2 * NCHUNK)
    for c in range(NCHUNK):
        gathers[c].wait()
        if c >= OB:
            writes[2 * (c - OB)].wait()
            writes[2 * (c - OB) + 1].wait()
        rv = rows[c % RB]
        re_v = res[c % OB]
        im_v = ims[c % OB]

        @plsc.parallel_loop(0, CHUNK, unroll=4)
        def body(r):
            row = jnp.full((16,), r, jnp.int32)
            vals = []
            for j in range(HALF // 16):
                vals.append(plsc.load_gather(rv, [row, cols[j]]))
                vals.append(plsc.load_gather(rv, [row, cols1[j]]))
            for j in range(HALF // 16):
                re_v[r, pl.ds(16 * j, 16)] = vals[2 * j]
                im_v[r, pl.ds(16 * j, 16)] = vals[2 * j + 1]

        if c + RB < NCHUNK:
            gathers[c + RB] = fire_gather(c + RB)
        off = base + c * CHUNK
        writes[2 * c] = pltpu.async_copy(
            re_v, re_hbm.at[pl.ds(off, CHUNK)], osem)
        writes[2 * c + 1] = pltpu.async_copy(
            im_v, im_hbm.at[pl.ds(off, CHUNK)], osem)

    for c in range(max(NCHUNK - OB, 0), NCHUNK):
        writes[2 * c].wait()
        writes[2 * c + 1].wait()


def kernel(input_ids, table):
    ids = input_ids.astype(jnp.int32)
    parts = [_gather_split(ids[i * BS:(i + 1) * BS], table)
             for i in range(NSPLIT)]
    real = jnp.concatenate([p[0] for p in parts], axis=0)
    imag = jnp.concatenate([p[1] for p in parts], axis=0)
    return real, imag


# speedup vs baseline: 1.3853x; 1.3853x over previous
import functools

import jax
import jax.numpy as jnp
from jax import lax
from jax.experimental import pallas as pl
from jax.experimental.pallas import tpu as pltpu
from jax.experimental.pallas import tpu_sc as plsc

NUM_EMB = 100000
D = 128
HALF = D // 2
B = 16384
NC = 2
NS = 16
NW = NC * NS
BPW = B // NW
CHUNK = 128
NCHUNK = BPW // CHUNK
RB = 2
OB = 2

_mesh = plsc.VectorSubcoreMesh(core_axis_name="c", subcore_axis_name="s")


@functools.partial(
    pl.kernel,
    mesh=_mesh,
    out_type=(jax.ShapeDtypeStruct((B, HALF), jnp.float32),
              jax.ShapeDtypeStruct((B, HALF), jnp.float32)),
    scratch_types=(
        [pltpu.VMEM((BPW,), jnp.int32)]
        + [pltpu.VMEM((CHUNK, D), jnp.float32) for _ in range(RB)]
        + [pltpu.VMEM((CHUNK, HALF), jnp.float32) for _ in range(2 * OB)]
        + [pltpu.SemaphoreType.DMA((NCHUNK,)), pltpu.SemaphoreType.DMA]
    ),
    compiler_params=pltpu.CompilerParams(needs_layout_passes=False),
)
def _gather_split(ids_hbm, table_hbm, re_hbm, im_hbm, idx_v,
                  rows0, rows1, re0, re1, im0, im1,
                  gsem, osem):
    rows = [rows0, rows1]
    res = [re0, re1]
    ims = [im0, im1]
    wid = lax.axis_index("s") * NC + lax.axis_index("c")
    base = wid * BPW
    pltpu.sync_copy(ids_hbm.at[pl.ds(base, BPW)], idx_v)

    def fire_gather(c):
        return pltpu.async_copy(
            table_hbm.at[idx_v.at[pl.ds(c * CHUNK, CHUNK)]],
            rows[c % RB], gsem.at[c])

    gathers = [None] * NCHUNK
    for c in range(RB):
        gathers[c] = fire_gather(c)

    evens = lax.iota(jnp.int32, 16) * 2
    cols = [evens + 32 * j for j in range(HALF // 16)]
    cols1 = [col + 1 for col in cols]

    writes = [None] * (2 * NCHUNK)
    for c in range(NCHUNK):
        gathers[c].wait()
        if c >= OB:
            writes[2 * (c - OB)].wait()
            writes[2 * (c - OB) + 1].wait()
        rv = rows[c % RB]
        re_v = res[c % OB]
        im_v = ims[c % OB]

        @plsc.parallel_loop(0, CHUNK, unroll=4)
        def body(r):
            row = jnp.full((16,), r, jnp.int32)
            vals = []
            for j in range(HALF // 16):
                vals.append(plsc.load_gather(rv, [row, cols[j]]))
                vals.append(plsc.load_gather(rv, [row, cols1[j]]))
            for j in range(HALF // 16):
                re_v[r, pl.ds(16 * j, 16)] = vals[2 * j]
                im_v[r, pl.ds(16 * j, 16)] = vals[2 * j + 1]

        if c + RB < NCHUNK:
            gathers[c + RB] = fire_gather(c + RB)
        off = base + c * CHUNK
        writes[2 * c] = pltpu.async_copy(
            re_v, re_hbm.at[pl.ds(off, CHUNK)], osem)
        writes[2 * c + 1] = pltpu.async_copy(
            im_v, im_hbm.at[pl.ds(off, CHUNK)], osem)

    for c in range(NCHUNK - OB, NCHUNK):
        writes[2 * c].wait()
        writes[2 * c + 1].wait()


def kernel(input_ids, table):
    return _gather_split(input_ids.astype(jnp.int32), table)
